# Initial kernel scaffold; baseline (speedup 1.0000x reference)
import jax
import jax.numpy as jnp
from jax.experimental import pallas as pl


def kernel(userIdx, servIdx, U_gmf, U_mlp, I_gmf, I_mlp, W0, b0, W1, b1, W2, b2, Wp, bp):
    raise NotImplementedError("write your pallas kernel here")


# speedup vs baseline: 1.9301x; 1.9301x over previous
import functools

import jax
import jax.numpy as jnp
from jax import lax
from jax.experimental import pallas as pl
from jax.experimental.pallas import tpu as pltpu
from jax.experimental.pallas import tpu_sc as plsc

BATCH = 16384
DIM = 64
DIM_MLP = 256
CHUNK = 128


def _make_gather(d, use_tc_tiling):
    info = plsc.get_sparse_core_info()
    nc, ns = info.num_cores, info.num_subcores
    nw = nc * ns
    b_per_w = BATCH // nw
    n_chunks = b_per_w // CHUNK
    mesh = plsc.VectorSubcoreMesh(core_axis_name="c", subcore_axis_name="s")

    f32 = jnp.float32

    @functools.partial(
        pl.kernel,
        mesh=mesh,
        out_type=[
            jax.ShapeDtypeStruct((BATCH, d), f32),
            jax.ShapeDtypeStruct((BATCH, d), f32),
        ],
        scratch_types=[
            pltpu.VMEM((n_chunks, CHUNK), jnp.int32),
            pltpu.VMEM((n_chunks, CHUNK), jnp.int32),
            pltpu.VMEM((CHUNK, d), f32),
            pltpu.VMEM((CHUNK, d), f32),
            pltpu.SemaphoreType.DMA,
            pltpu.SemaphoreType.DMA,
        ],
        compiler_params=pltpu.CompilerParams(use_tc_tiling_on_sc=use_tc_tiling),
    )
    def gather_kernel(u_idx_hbm, s_idx_hbm, ut_hbm, it_hbm,
                      out_u, out_i,
                      idx_u, idx_i, buf_a, buf_b, sem_a, sem_b):
        wid = lax.axis_index("s") * nc + lax.axis_index("c")
        base = wid * b_per_w
        for j in range(n_chunks):
            pltpu.sync_copy(u_idx_hbm.at[pl.ds(base + j * CHUNK, CHUNK)],
                            idx_u.at[j])
            pltpu.sync_copy(s_idx_hbm.at[pl.ds(base + j * CHUNK, CHUNK)],
                            idx_i.at[j])
        for j in range(n_chunks):
            row0 = base + j * CHUNK
            pltpu.async_copy(ut_hbm.at[idx_u.at[j]], buf_a, sem_a).wait()
            pltpu.sync_copy(buf_a, out_u.at[pl.ds(row0, CHUNK)])
            pltpu.async_copy(it_hbm.at[idx_i.at[j]], buf_b, sem_b).wait()
            pltpu.sync_copy(buf_b, out_i.at[pl.ds(row0, CHUNK)])

    return gather_kernel


_gather_mlp = _make_gather(DIM_MLP, True)
_gather_gmf = _make_gather(DIM, False)


def _mlp_body(ug, ig, um, im, w0u, w0i, b0, w1, b1, w2, b2, wpg, wpx, bp, out):
    f32 = jnp.float32
    x = jnp.dot(um[...], w0u[...], preferred_element_type=f32)
    x = x + jnp.dot(im[...], w0i[...], preferred_element_type=f32)
    x = jnp.maximum(x + b0[...], 0.0)
    x = jnp.maximum(jnp.dot(x, w1[...], preferred_element_type=f32) + b1[...], 0.0)
    x = jnp.maximum(jnp.dot(x, w2[...], preferred_element_type=f32) + b2[...], 0.0)
    g = ug[...] * ig[...]
    pred = (jnp.dot(g, wpg[...], preferred_element_type=f32)
            + jnp.dot(x, wpx[...], preferred_element_type=f32)
            + bp[...])
    out[...] = pred


def _run_mlp(ug, ig, um, im, W0, b0, W1, b1, W2, b2, Wp, bp):
    T = 1024
    grid = (BATCH // T,)
    f32 = jnp.float32
    w0u = W0[:, :DIM_MLP].T
    w0i = W0[:, DIM_MLP:].T
    w1 = W1.T
    w2 = W2.T
    wpg = Wp[:, :DIM].T
    wpx = Wp[:, DIM:].T
    b0r = b0.reshape(1, -1)
    b1r = b1.reshape(1, -1)
    b2r = b2.reshape(1, -1)
    bpr = bp.reshape(1, 1)

    batch_spec = lambda d: pl.BlockSpec((T, d), lambda i: (i, 0))
    full_spec = lambda a, b: pl.BlockSpec((a, b), lambda i: (0, 0))

    out = pl.pallas_call(
        _mlp_body,
        grid=grid,
        in_specs=[
            batch_spec(DIM), batch_spec(DIM), batch_spec(DIM_MLP), batch_spec(DIM_MLP),
            full_spec(256, 256), full_spec(256, 256), full_spec(1, 256),
            full_spec(256, 128), full_spec(1, 128),
            full_spec(128, 64), full_spec(1, 64),
            full_spec(64, 1), full_spec(64, 1), full_spec(1, 1),
        ],
        out_specs=pl.BlockSpec((T, 1), lambda i: (i, 0)),
        out_shape=jax.ShapeDtypeStruct((BATCH, 1), f32),
    )(ug, ig, um, im, w0u, w0i, b0r, w1, b1r, w2, b2r, wpg, wpx, bpr)
    return out.reshape(-1)


def kernel(userIdx, servIdx, U_gmf, U_mlp, I_gmf, I_mlp, W0, b0, W1, b1, W2, b2, Wp, bp):
    ui = userIdx.astype(jnp.int32)
    si = servIdx.astype(jnp.int32)
    um, im = _gather_mlp(ui, si, U_mlp, I_mlp)
    ug, ig = _gather_gmf(ui, si, U_gmf, I_gmf)
    return _run_mlp(ug, ig, um, im, W0, b0, W1, b1, W2, b2, Wp, bp)
